# Initial kernel scaffold; baseline (speedup 1.0000x reference)
import jax
import jax.numpy as jnp
from jax.experimental import pallas as pl


def kernel(x, edge_index, edge_type, batch, W1, root1, b1, W2, root2, b2, W3, root3, b3, Wc1, bc1, Wc2, bc2):
    raise NotImplementedError("write your pallas kernel here")


# speedup vs baseline: 5.8081x; 5.8081x over previous
import functools

import jax
import jax.numpy as jnp
from jax import lax
from jax.experimental import pallas as pl
from jax.experimental.pallas import tpu as pltpu
from jax.experimental.pallas import tpu_sc as plsc

N = 100000
E = 1600000
NREL = 2
NGRAPH = 64
IN_CH = 8
HID = 64

NTILE = 16
SUB = 128
KBLK = 2048
NSUB = KBLK // SUB
EPT = 100352
EPAD = NTILE * EPT
NBLK = EPT // KBLK
E2ROWS = EPAD // SUB
ACC_ROWS = 204800
TROWS = ACC_ROWS // NTILE
TRASH = 2 * N
ZROWS = 512


def _make_seg_sum(G):
  Gsc = G // 2
  mesh = plsc.VectorSubcoreMesh(core_axis_name="c", subcore_axis_name="s")

  @functools.partial(
      pl.kernel,
      out_type=jax.ShapeDtypeStruct((G * ACC_ROWS, 8), jnp.float32),
      mesh=mesh,
      scratch_types=[
          pltpu.VMEM_SHARED((ACC_ROWS, 8), jnp.float32),
          pltpu.VMEM((ZROWS, 8), jnp.float32),
          pltpu.VMEM((NSUB, SUB), jnp.int32),
          pltpu.VMEM((NSUB, SUB), jnp.int32),
          pltpu.VMEM((KBLK, 8), jnp.float32),
          pltpu.SemaphoreType.DMA,
          pltpu.SemaphoreType.DMA,
      ],
      compiler_params=pltpu.CompilerParams(use_tc_tiling_on_sc=False),
  )
  def seg_sum(tab_hbm, src_hbm, key_hbm, zero_hbm, out_hbm,
              acc, zbuf, srcv, keyv, rows, gsem, ssem):
    c = lax.axis_index("c")
    s = lax.axis_index("s")
    pltpu.sync_copy(zero_hbm, zbuf)
    for k in range(TROWS // ZROWS):
      pltpu.sync_copy(zbuf, acc.at[pl.ds(s * TROWS + k * ZROWS, ZROWS)])
    plsc.subcore_barrier()

    for p in range(Gsc):
      g = c * Gsc + p

      def body(i, _):
        base2 = g * E2ROWS + s * (EPT // SUB) + i * NSUB
        pltpu.sync_copy(src_hbm.at[pl.ds(base2, NSUB)], srcv)
        kbase2 = s * (EPT // SUB) + i * NSUB
        pltpu.sync_copy(key_hbm.at[pl.ds(kbase2, NSUB)], keyv)
        descs = []
        for j in range(NSUB):
          descs.append(pltpu.async_copy(
              tab_hbm.at[srcv.at[j]], rows.at[pl.ds(j * SUB, SUB)], gsem))
        for d in descs:
          d.wait()
        descs = []
        for j in range(NSUB):
          descs.append(pltpu.async_copy(
              rows.at[pl.ds(j * SUB, SUB)], acc.at[keyv.at[j]], ssem,
              add=True))
        for d in descs:
          d.wait()
        return 0

      lax.fori_loop(0, NBLK, body, 0)
      plsc.subcore_barrier()
      pltpu.sync_copy(acc.at[pl.ds(s * TROWS, TROWS)],
                      out_hbm.at[pl.ds(g * ACC_ROWS + s * TROWS, TROWS)])
      for k in range(TROWS // ZROWS):
        pltpu.sync_copy(zbuf, acc.at[pl.ds(s * TROWS + k * ZROWS, ZROWS)])
      plsc.subcore_barrier()

  return seg_sum


RB = 800
NB = N // RB
SB = ACC_ROWS // RB


def _sspec(g, r):
  return pl.BlockSpec((RB, 8), lambda i, g=g, r=r: (g * SB + r * NB + i, 0))


def _full(shape):
  return pl.BlockSpec(shape, lambda i: tuple(0 for _ in shape))


def _dense1(x, s1, root, w, b):
  def body(x_ref, s0_ref, s1_ref, c0_ref, c1_ref, root_ref, w_ref, b_ref,
           out_ref):
    xb = x_ref[...]
    out = jnp.dot(xb, root_ref[...], preferred_element_type=jnp.float32)
    out = out + b_ref[...]
    for r, (s_ref, c_ref) in enumerate(((s0_ref, c0_ref), (s1_ref, c1_ref))):
      inv = 1.0 / jnp.maximum(c_ref[...][:, 0:1], 1.0)
      agg = jnp.dot(s_ref[...], w_ref[r], preferred_element_type=jnp.float32)
      out = out + agg * inv
    out = jnp.maximum(out, 0.0)
    for gg in range(8):
      out_ref[gg] = out[:, gg * 8:(gg + 1) * 8]

  return pl.pallas_call(
      body,
      grid=(NB,),
      in_specs=[
          pl.BlockSpec((RB, 8), lambda i: (i, 0)),
          _sspec(0, 0), _sspec(0, 1), _sspec(1, 0), _sspec(1, 1),
          _full((8, 64)), _full((2, 8, 64)), _full((1, 64)),
      ],
      out_specs=pl.BlockSpec((8, RB, 8), lambda i: (0, i, 0)),
      out_shape=jax.ShapeDtypeStruct((8, N, 8), jnp.float32),
  )(x, s1, s1, s1, s1, root, w, b)


def _dense23(h8, sl, s1, root, w, b, relu, batch2, wc1, bc1, wc2, bc2):
  ns = 16

  def body(*refs):
    h_ref = refs[0]
    s_refs = refs[1:1 + ns]
    c_refs = refs[1 + ns:3 + ns]
    root_ref, w_ref, b_ref = refs[3 + ns:6 + ns]
    if relu:
      out_ref = refs[6 + ns]
      scratch = ()
    else:
      batch_ref, wc1_ref, bc1_ref, wc2_ref, bc2_ref = refs[6 + ns:11 + ns]
      out_ref = refs[11 + ns]
      gacc = refs[12 + ns]
    hb = jnp.concatenate([h_ref[gg] for gg in range(8)], axis=1)
    out = jnp.dot(hb, root_ref[...], preferred_element_type=jnp.float32)
    out = out + b_ref[...]
    for r in range(2):
      inv = 1.0 / jnp.maximum(c_refs[r][...][:, 0:1], 1.0)
      agg = jnp.zeros((RB, HID), jnp.float32)
      for gg in range(8):
        agg = agg + jnp.dot(s_refs[gg * 2 + r][...],
                            w_ref[r, gg * 8:(gg + 1) * 8, :],
                            preferred_element_type=jnp.float32)
      out = out + agg * inv
    if relu:
      out = jnp.maximum(out, 0.0)
      for gg in range(8):
        out_ref[gg] = out[:, gg * 8:(gg + 1) * 8]
    else:
      i = pl.program_id(0)
      bvals = batch_ref[...]
      iot = lax.broadcasted_iota(jnp.int32, (RB, NGRAPH), 1)
      oh = (bvals == iot).astype(jnp.float32)
      pg = lax.dot_general(oh, out, (((0,), (0,)), ((), ())),
                           preferred_element_type=jnp.float32)

      @pl.when(i == 0)
      def _():
        gacc[...] = jnp.zeros((NGRAPH, HID), jnp.float32)

      gacc[...] += pg

      @pl.when(i == NB - 1)
      def _():
        gv = gacc[...]
        hcl = jnp.maximum(
            jnp.dot(gv, wc1_ref[...], preferred_element_type=jnp.float32)
            + bc1_ref[...], 0.0)
        out_ref[...] = (jnp.dot(hcl, wc2_ref[...],
                                preferred_element_type=jnp.float32)
                        + bc2_ref[...])

  in_specs = [pl.BlockSpec((8, RB, 8), lambda i: (0, i, 0))]
  args = [h8]
  for gg in range(8):
    for r in range(2):
      in_specs.append(_sspec(gg, r))
      args.append(sl)
  in_specs += [_sspec(1, 0), _sspec(1, 1)]
  args += [s1, s1]
  in_specs += [_full((HID, HID)), _full((2, HID, HID)), _full((1, HID))]
  args += [root, w, b]
  if relu:
    out_specs = pl.BlockSpec((8, RB, 8), lambda i: (0, i, 0))
    out_shape = jax.ShapeDtypeStruct((8, N, 8), jnp.float32)
    scratch_shapes = []
  else:
    in_specs += [pl.BlockSpec((RB, 1), lambda i: (i, 0)),
                 _full((HID, HID // 2)), _full((1, HID // 2)),
                 _full((HID // 2, 1)), _full((1, 1))]
    args += [batch2, wc1, bc1, wc2, bc2]
    out_specs = pl.BlockSpec((NGRAPH, 1), lambda i: (0, 0))
    out_shape = jax.ShapeDtypeStruct((NGRAPH, 1), jnp.float32)
    scratch_shapes = [pltpu.VMEM((NGRAPH, HID), jnp.float32)]

  return pl.pallas_call(
      body,
      grid=(NB,),
      in_specs=in_specs,
      out_specs=out_specs,
      out_shape=out_shape,
      scratch_shapes=scratch_shapes,
  )(*args)


def kernel(x, edge_index, edge_type, batch, W1, root1, b1, W2, root2, b2,
           W3, root3, b3, Wc1, bc1, Wc2, bc2):
  x = x.astype(jnp.float32)
  src = edge_index[0].astype(jnp.int32)
  dst = edge_index[1].astype(jnp.int32)
  et = edge_type.astype(jnp.int32)

  pad = EPAD - E
  key = et * N + dst
  key_pad = jnp.concatenate([key, jnp.full((pad,), TRASH, jnp.int32)])
  key2 = key_pad.reshape(E2ROWS, SUB)
  src_pad = jnp.concatenate([src, jnp.zeros((pad,), jnp.int32)])
  src_all = (src_pad[None, :]
             + (jnp.arange(8, dtype=jnp.int32) * N)[:, None])
  src8 = src_all.reshape(8 * E2ROWS, SUB)
  src2 = src_all[:2].reshape(2 * E2ROWS, SUB)
  zeros_z = jnp.zeros((ZROWS, 8), jnp.float32)
  ones_tab = jnp.concatenate(
      [jnp.ones((N, 1), jnp.float32), jnp.zeros((N, 7), jnp.float32)], axis=1)
  tab1 = jnp.concatenate([x, ones_tab], axis=0)
  batch2 = batch.astype(jnp.int32).reshape(N, 1)
  b1r = b1.reshape(1, HID)
  b2r = b2.reshape(1, HID)
  b3r = b3.reshape(1, HID)
  bc1r = bc1.reshape(1, HID // 2)
  bc2r = bc2.reshape(1, 1)

  seg2 = _make_seg_sum(2)
  seg8 = _make_seg_sum(8)

  s1 = seg2(tab1, src2, key2, zeros_z)
  h1 = _dense1(x, s1, root1, W1, b1r)
  s2 = seg8(h1.reshape(8 * N, 8), src8, key2, zeros_z)
  h2 = _dense23(h1, s2, s1, root2, W2, b2r, True,
                None, None, None, None, None)
  s3 = seg8(h2.reshape(8 * N, 8), src8, key2, zeros_z)
  out = _dense23(h2, s3, s1, root3, W3, b3r, False,
                 batch2, Wc1, bc1r, Wc2, bc2r)
  return out
